# baseline (speedup 1.0000x reference)
import functools

import jax
import jax.numpy as jnp
from jax import lax
from jax.experimental import pallas as pl
from jax.experimental.pallas import tpu as pltpu
from jax.experimental.pallas import tpu_sc as plsc

NC = 2
NS = 16
NW = NC * NS
LANES = 16
WIN = 128
GW = 4
RING = 128


@functools.lru_cache(maxsize=None)
def _make_emb(b, v_rows, embed_dim):
  mesh = plsc.VectorSubcoreMesh(core_axis_name="c", subcore_axis_name="s")
  n_vecs = b // LANES
  n_rows = embed_dim // LANES
  n_win = (v_rows + WIN - 1) // WIN
  n_groups = (n_win + GW - 1) // GW
  max_bins = n_groups // NW + 2

  @functools.partial(
      pl.kernel,
      mesh=mesh,
      compiler_params=pltpu.CompilerParams(needs_layout_passes=False),
      out_type=jax.ShapeDtypeStruct((b + NW, WIN), jnp.float32),
      scratch_types=[
          pltpu.VMEM((b,), jnp.int32),
          pltpu.VMEM((b,), jnp.int32),
          pltpu.VMEM((b,), jnp.int32),
          pltpu.VMEM((max_bins * LANES,), jnp.int32),
          pltpu.VMEM((max_bins * LANES,), jnp.int32),
          pltpu.VMEM((2, GW, embed_dim, WIN), jnp.float32),
          pltpu.VMEM((2 * RING, WIN), jnp.float32),
          pltpu.VMEM((2, RING), jnp.int32),
          pltpu.SemaphoreType.DMA,
          pltpu.SemaphoreType.DMA,
      ],
  )
  def emb(idx_hbm, tbl_t_hbm, out_hbm, idx_v, mr, mp, starts, bases,
          win_v, rows_v, ring_pos, sem_a, sem_b):
    wid = lax.axis_index("s") * NC + lax.axis_index("c")
    lo_g = (wid * n_groups) // NW
    hi_g = ((wid + 1) * n_groups) // NW
    dump = b + wid
    iota = lax.iota(jnp.int32, LANES)
    row_iota = [iota + k * LANES for k in range(n_rows)]

    pltpu.sync_copy(idx_hbm, idx_v)

    for k in range(max_bins):
      bases[pl.ds(k * LANES, LANES)] = jnp.zeros((LANES,), jnp.int32)
    for p in range(2):
      for k in range(RING // LANES):
        plsc.store_scatter(
            ring_pos,
            [jnp.broadcast_to(p, (LANES,)), iota + k * LANES],
            jnp.broadcast_to(dump, (LANES,)),
        )

    def hist_body(v):
      vec = idx_v[pl.ds(v * LANES, LANES)]
      g = lax.shift_right_logical(vec, 7 + GW.bit_length() - 1)
      m = (g >= lo_g) & (g < hi_g)
      bl = g - lo_g
      rank, last = plsc.scan_count(bl, mask=m)
      plsc.addupdate_scatter(bases, [bl], rank, mask=m & last)

    pl.loop(0, n_vecs, unroll=8)(hist_body)

    def scan_bins(k, run):
      h = bases[pl.ds(k * LANES, LANES)]
      c = plsc.cumsum(h)
      excl = c - h + run
      starts[pl.ds(k * LANES, LANES)] = excl
      bases[pl.ds(k * LANES, LANES)] = excl
      return run + c[LANES - 1]

    pl.loop(0, max_bins, init_carry=jnp.int32(0))(scan_bins)

    def fill_body(v):
      vec = idx_v[pl.ds(v * LANES, LANES)]
      g = lax.shift_right_logical(vec, 7 + GW.bit_length() - 1)
      m = (g >= lo_g) & (g < hi_g)
      bl = g - lo_g
      rank, last = plsc.scan_count(bl, mask=m)
      rank0 = rank - 1
      base_v = plsc.load_gather(bases, [bl])
      slot = base_v + rank0
      plsc.store_scatter(mr, [slot], vec, mask=m)
      plsc.store_scatter(mp, [slot], iota + v * LANES, mask=m)
      plsc.addupdate_scatter(bases, [bl], rank, mask=m & last)

    pl.loop(0, n_vecs, unroll=8)(fill_body)

    def fetch(g, buf, sem):
      for j in range(GW):
        w = jnp.minimum(g * GW + j, n_win - 1)
        lane = pl.multiple_of(lax.shift_left(w, 7), WIN)
        pltpu.async_copy(
            tbl_t_hbm.at[:, pl.ds(lane, WIN)], win_v.at[buf, j], sem)

    def drain(buf, sem):
      for j in range(GW):
        pltpu.make_async_copy(
            tbl_t_hbm.at[:, pl.ds(0, WIN)], win_v.at[buf, j], sem).wait()

    def extract_group(g, buf, rc):
      gl = jnp.minimum(g - lo_g, max_bins * LANES - 1)
      bstart = plsc.load_gather(starts, [jnp.broadcast_to(gl, (LANES,))])[0]
      bend = plsc.load_gather(bases, [jnp.broadcast_to(gl, (LANES,))])[0]

      def mv_body(v, rc):
        at = bstart + v * LANES
        vr = mr[pl.ds(at, LANES)]
        vp = mp[pl.ds(at, LANES)]
        nv = jnp.minimum(bend - at, LANES)
        valid = iota < nv
        sl = rc + iota
        plsc.store_scatter(
            ring_pos,
            [lax.shift_right_logical(sl, 7) & 1, sl & (RING - 1)],
            vp, mask=valid)
        for l in range(LANES):
          @pl.when(l < nv)
          def _one():
            r = vr[l]
            jw = lax.shift_right_logical(r, 7) - g * GW
            col = jnp.broadcast_to(r & (WIN - 1), (LANES,))
            s = (rc + l) & (2 * RING - 1)
            for k in range(n_rows):
              vals = plsc.load_gather(
                  win_v.at[buf],
                  [jnp.broadcast_to(jw, (LANES,)), row_iota[k], col])
              plsc.store_scatter(
                  rows_v, [jnp.broadcast_to(s, (LANES,)), row_iota[k]], vals)
        rc2 = rc + nv

        @pl.when(
            lax.shift_right_logical(rc2, 7) != lax.shift_right_logical(rc, 7))
        def _flush():
          p = lax.shift_right_logical(rc, 7) & 1
          off = pl.multiple_of(p * RING, RING)
          pltpu.sync_copy(
              rows_v.at[pl.ds(off, RING)], out_hbm.at[ring_pos.at[p]])
          for k in range(RING // LANES):
            plsc.store_scatter(
                ring_pos,
                [jnp.broadcast_to(p, (LANES,)), iota + k * LANES],
                jnp.broadcast_to(dump, (LANES,)))

        return rc2

      nmv = lax.shift_right_logical(bend - bstart + LANES - 1, 4)
      return pl.loop(0, nmv, init_carry=rc)(mv_body)

    n_my = hi_g - lo_g
    fetch(lo_g, 0, sem_a)

    def pair_body(t, rc):
      g0 = lo_g + t
      fetch(jnp.minimum(g0 + 1, hi_g - 1), 1, sem_b)
      drain(0, sem_a)
      rc = extract_group(g0, 0, rc)
      fetch(jnp.minimum(g0 + 2, hi_g - 1), 0, sem_a)
      drain(1, sem_b)
      return extract_group(g0 + 1, 1, rc)

    rc_fin = pl.loop(0, n_my, step=2, init_carry=jnp.int32(0))(pair_body)
    drain(0, sem_a)

    for p in range(2):
      pltpu.sync_copy(
          rows_v.at[pl.ds(p * RING, RING)], out_hbm.at[ring_pos.at[p]])
    del rc_fin

  return emb


@jax.jit
def kernel(cls_idx, table):
  b = cls_idx.shape[0]
  v_rows, embed_dim = table.shape
  idx = cls_idx.astype(jnp.int32)
  out_pad = _make_emb(b, v_rows, embed_dim)(idx, table.T)
  return out_pad[:b, :embed_dim].reshape(b, 1, embed_dim)


# speedup vs baseline: 1.1382x; 1.1382x over previous
import functools

import jax
import jax.numpy as jnp
from jax import lax
from jax.experimental import pallas as pl
from jax.experimental.pallas import tpu as pltpu
from jax.experimental.pallas import tpu_sc as plsc

NC = 2
NS = 16
NW = NC * NS
LANES = 16
WIN = 128


@functools.lru_cache(maxsize=None)
def _make_emb(b, embed_dim):
  mesh = plsc.VectorSubcoreMesh(core_axis_name="c", subcore_axis_name="s")
  b_per_w = b // NW
  n_groups = b_per_w // LANES
  n_rows = embed_dim // LANES

  @functools.partial(
      pl.kernel,
      mesh=mesh,
      compiler_params=pltpu.CompilerParams(needs_layout_passes=False),
      out_type=jax.ShapeDtypeStruct((embed_dim, b), jnp.float32),
      scratch_types=[
          pltpu.VMEM((b_per_w,), jnp.int32),
          pltpu.VMEM((LANES, embed_dim, WIN), jnp.float32),
          pltpu.VMEM((embed_dim, WIN), jnp.float32),
          pltpu.SemaphoreType.DMA,
      ],
  )
  def emb(idx_hbm, tbl_t_hbm, out_hbm, idx_v, win_v, dims_v, sem):
    wid = lax.axis_index("s") * NC + lax.axis_index("c")
    base = wid * b_per_w
    pltpu.sync_copy(idx_hbm.at[wid], idx_v)
    row_iota = [
        lax.iota(jnp.int32, LANES) + k * LANES for k in range(n_rows)
    ]

    def body(g):
      vec = idx_v[pl.ds(g * LANES, LANES)]
      copies = []
      for l in range(LANES):
        lane_base = pl.multiple_of((vec[l] >> 7) << 7, WIN)
        copies.append(
            pltpu.async_copy(
                tbl_t_hbm.at[:, pl.ds(lane_base, WIN)], win_v.at[l], sem))
      col_in_group = (g & 7) * LANES
      for l in range(LANES):
        copies[l].wait()
        col = jnp.broadcast_to(vec[l] & 127, (LANES,))
        pos = jnp.broadcast_to(col_in_group + l, (LANES,))
        for k in range(n_rows):
          vals = plsc.load_gather(win_v.at[l], [row_iota[k], col])
          plsc.store_scatter(dims_v, [row_iota[k], pos], vals)

      @pl.when((g & 7) == 7)
      def _flush():
        out_base = pl.multiple_of(base + ((g >> 3) << 7), WIN)
        pltpu.sync_copy(dims_v, out_hbm.at[:, pl.ds(out_base, WIN)])

    pl.loop(0, n_groups)(body)

  return emb


@jax.jit
def kernel(cls_idx, table):
  b = cls_idx.shape[0]
  embed_dim = table.shape[1]
  idx = cls_idx.astype(jnp.int32).reshape(NW, b // NW)
  out_t = _make_emb(b, embed_dim)(idx, table.T)
  return out_t.T.reshape(b, 1, embed_dim)
